# baseline (speedup 1.0000x reference)
import functools

import jax
import jax.numpy as jnp
from jax import lax
from jax.experimental import pallas as pl
from jax.experimental.pallas import tpu as pltpu
from jax.experimental.pallas import tpu_sc as plsc

N = 10000
E = 320000
D = 128

NC, NS = 2, 16
NW = NC * NS
CH = 64
CPW = 160
SCH = 40
NBUF = 4
NCHUNK = NW * CPW
EPAD = NCHUNK * CH
ACC_ROWS = 10240
PAD_DST = N
RPW = ACC_ROWS // NS

MM_BLK = 1000


def _matmul(x, w):
    def body(x_ref, w_ref, o_ref):
        o_ref[...] = jnp.dot(x_ref[...], w_ref[...],
                             preferred_element_type=jnp.float32)

    return pl.pallas_call(
        body,
        grid=(N // MM_BLK,),
        in_specs=[
            pl.BlockSpec((MM_BLK, D), lambda i: (i, 0)),
            pl.BlockSpec((D, D), lambda i: (0, 0)),
        ],
        out_specs=pl.BlockSpec((MM_BLK, D), lambda i: (i, 0)),
        out_shape=jax.ShapeDtypeStruct((N, D), jnp.float32),
    )(x, w)


def _sc_scatter(h0, h1, h2, src, dst):
    mesh = plsc.VectorSubcoreMesh(core_axis_name="c", subcore_axis_name="s")

    @functools.partial(
        pl.kernel,
        out_type=jax.ShapeDtypeStruct((3, NC, ACC_ROWS, D), jnp.float32),
        mesh=mesh,
        scratch_types=[
            pltpu.VMEM((SCH, CH), jnp.int32),
            pltpu.VMEM((SCH, CH), jnp.int32),
            [pltpu.VMEM((CH, D), jnp.float32) for _ in range(NBUF)],
            pltpu.VMEM_SHARED((ACC_ROWS, D), jnp.float32),
            [pltpu.SemaphoreType.DMA for _ in range(NBUF)],
            [pltpu.SemaphoreType.DMA for _ in range(NBUF)],
        ],
    )
    def k(h0_hbm, h1_hbm, h2_hbm, src_hbm, dst_hbm, p_hbm,
          srcv, dstv, rows, acc, sems, ssems):
        c = lax.axis_index("c")
        s = lax.axis_index("s")
        wid = c * NS + s
        hs = (h0_hbm, h1_hbm, h2_hbm)

        def fill_zero():
            def fb(r, carry):
                for j in range(D // 16):
                    rows[0][r, pl.ds(j * 16, 16)] = jnp.zeros((16,), jnp.float32)
                return carry
            lax.fori_loop(0, CH, fb, 0)

        def zero_slice():
            for j in range(RPW // CH):
                pltpu.sync_copy(rows[0], acc.at[pl.ds(s * RPW + j * CH, CH)])

        def edge_phase(g):
            h = hs[g]

            def gather(ci, b):
                pltpu.async_copy(h.at[srcv.at[ci]], rows[b], sems[b])

            def gather_wait(ci, b):
                pltpu.make_async_copy(h.at[srcv.at[ci]], rows[b], sems[b]).wait()

            def scatter(ci, b):
                pltpu.async_copy(rows[b], acc.at[dstv.at[ci]], ssems[b], add=True)

            def scatter_wait(ci, b):
                pltpu.make_async_copy(rows[b], acc.at[dstv.at[ci]], ssems[b]).wait()

            def slot(ci, b, first=False, fire=True):
                gather_wait(ci, b)
                scatter(ci, b)
                bn = (b + NBUF - 1) % NBUF
                if not first:
                    scatter_wait(ci - 1, bn)
                if fire:
                    gather(ci + NBUF - 1, bn)

            for strip in range(CPW // SCH):
                sbase = wid * CPW + strip * SCH
                pltpu.sync_copy(src_hbm.at[g, pl.ds(sbase, SCH)], srcv)
                pltpu.sync_copy(dst_hbm.at[g, pl.ds(sbase, SCH)], dstv)

                for b in range(NBUF - 1):
                    gather(b, b)
                slot(0, 0, first=True)
                for ci in range(1, NBUF):
                    slot(ci, ci % NBUF)

                def body(o, carry):
                    for b in range(NBUF):
                        slot(o * NBUF + b, b)
                    return carry

                lax.fori_loop(1, SCH // NBUF - 1, body, 0)
                for ci in range(SCH - NBUF, SCH):
                    slot(ci, ci % NBUF, fire=ci + NBUF - 1 < SCH)
                scatter_wait(SCH - 1, (SCH - 1) % NBUF)

        fill_zero()
        zero_slice()
        plsc.subcore_barrier()

        for g in range(3):
            edge_phase(g)
            plsc.subcore_barrier()
            pltpu.sync_copy(acc.at[pl.ds(s * RPW, RPW)],
                            p_hbm.at[g, c, pl.ds(s * RPW, RPW)])
            if g < 2:
                fill_zero()
                zero_slice()
            plsc.subcore_barrier()

    return k(h0, h1, h2, src, dst)


def _epilogue(p, g, x, b):
    def body(p0_ref, p1_ref, x_ref, b_ref, o_ref):
        agg = p0_ref[0, 0] + p1_ref[0, 0] + b_ref[...]
        o_ref[...] = jnp.maximum(agg, 0.0) + x_ref[...]

    return pl.pallas_call(
        body,
        grid=(N // MM_BLK,),
        in_specs=[
            pl.BlockSpec((1, 1, MM_BLK, D), lambda i, g=g: (g, 0, i, 0)),
            pl.BlockSpec((1, 1, MM_BLK, D), lambda i, g=g: (g, 1, i, 0)),
            pl.BlockSpec((MM_BLK, D), lambda i: (i, 0)),
            pl.BlockSpec((1, D), lambda i: (0, 0)),
        ],
        out_specs=pl.BlockSpec((MM_BLK, D), lambda i: (i, 0)),
        out_shape=jax.ShapeDtypeStruct((N, D), jnp.float32),
    )(p, p, x, b)


def _prep_edges(edge_index):
    src = edge_index[0].astype(jnp.int32)
    dst = edge_index[1].astype(jnp.int32)
    pad = jnp.arange(EPAD - E, dtype=jnp.int32)
    pad_src = (pad * 197) % N
    pad_dst = PAD_DST + pad % (ACC_ROWS - N)
    src = jnp.concatenate([src, pad_src])
    dst = jnp.concatenate([dst, pad_dst])
    return src.reshape(NCHUNK, CH), dst.reshape(NCHUNK, CH)


def kernel(h_mf_new, h_bp_new, h_cc_new, mf_edge_index, bp_edge_index,
           cc_edge_index, W_mf, b_mf, W_bp, b_bp, W_cc, b_cc):
    xs = (h_mf_new, h_bp_new, h_cc_new)
    hs = tuple(_matmul(x, w) for x, w in zip(xs, (W_mf, W_bp, W_cc)))

    se, de = zip(*(_prep_edges(e) for e in
                   (mf_edge_index, bp_edge_index, cc_edge_index)))
    src = jnp.stack(se)
    dst = jnp.stack(de)

    p = _sc_scatter(hs[0], hs[1], hs[2], src, dst)

    bs = (b_mf, b_bp, b_cc)
    outs = tuple(_epilogue(p, g, xs[g], bs[g].reshape(1, D)) for g in range(3))
    return outs


# speedup vs baseline: 1.0224x; 1.0224x over previous
import functools

import jax
import jax.numpy as jnp
from jax import lax
from jax.experimental import pallas as pl
from jax.experimental.pallas import tpu as pltpu
from jax.experimental.pallas import tpu_sc as plsc

N = 10000
E = 320000
D = 128

NC, NS = 2, 16
NW = NC * NS
CH = 64
CPW = 160
SCH = 40
NBUF = 4
NCHUNK = NW * CPW
EPAD = NCHUNK * CH
ACC_ROWS = 10240
PAD_DST = N
RPW = ACC_ROWS // NS

MM_BLK = 1000


def _matmul(x, w):
    def body(x_ref, w_ref, o_ref):
        o_ref[...] = jnp.dot(x_ref[...], w_ref[...],
                             preferred_element_type=jnp.float32)

    return pl.pallas_call(
        body,
        grid=(N // MM_BLK,),
        in_specs=[
            pl.BlockSpec((MM_BLK, D), lambda i: (i, 0)),
            pl.BlockSpec((D, D), lambda i: (0, 0)),
        ],
        out_specs=pl.BlockSpec((MM_BLK, D), lambda i: (i, 0)),
        out_shape=jax.ShapeDtypeStruct((N, D), jnp.float32),
    )(x, w)


def _sc_scatter(h0, h1, h2, srcs, dsts):
    mesh = plsc.VectorSubcoreMesh(core_axis_name="c", subcore_axis_name="s")

    @functools.partial(
        pl.kernel,
        out_type=jax.ShapeDtypeStruct((3, NC, ACC_ROWS, D), jnp.float32),
        mesh=mesh,
        scratch_types=[
            pltpu.VMEM((SCH, CH), jnp.int32),
            pltpu.VMEM((SCH, CH), jnp.int32),
            [pltpu.VMEM((CH, D), jnp.float32) for _ in range(NBUF)],
            pltpu.VMEM_SHARED((ACC_ROWS, D), jnp.float32),
            [pltpu.SemaphoreType.DMA for _ in range(NBUF)],
            [pltpu.SemaphoreType.DMA for _ in range(NBUF)],
        ],
    )
    def k(h0_hbm, h1_hbm, h2_hbm, s0_hbm, s1_hbm, s2_hbm, d0_hbm, d1_hbm,
          d2_hbm, p_hbm, srcv, dstv, rows, acc, sems, ssems):
        c = lax.axis_index("c")
        s = lax.axis_index("s")
        wid = c * NS + s
        hs = (h0_hbm, h1_hbm, h2_hbm)
        srcs_hbm = (s0_hbm, s1_hbm, s2_hbm)
        dsts_hbm = (d0_hbm, d1_hbm, d2_hbm)

        def fill_zero():
            def fb(r, carry):
                for j in range(D // 16):
                    rows[0][r, pl.ds(j * 16, 16)] = jnp.zeros((16,), jnp.float32)
                return carry
            lax.fori_loop(0, CH, fb, 0)

        def zero_slice():
            for j in range(RPW // CH):
                pltpu.sync_copy(rows[0], acc.at[pl.ds(s * RPW + j * CH, CH)])

        def edge_phase(g):
            h = hs[g]

            def gather(ci, b):
                pltpu.async_copy(h.at[srcv.at[ci]], rows[b], sems[b])

            def gather_wait(ci, b):
                pltpu.make_async_copy(h.at[srcv.at[ci]], rows[b], sems[b]).wait()

            def scatter(ci, b):
                pltpu.async_copy(rows[b], acc.at[dstv.at[ci]], ssems[b], add=True)

            def scatter_wait(ci, b):
                pltpu.make_async_copy(rows[b], acc.at[dstv.at[ci]], ssems[b]).wait()

            def slot(ci, b, first=False, fire=True):
                gather_wait(ci, b)
                scatter(ci, b)
                bn = (b + NBUF - 1) % NBUF
                if not first:
                    scatter_wait(ci - 1, bn)
                if fire:
                    gather(ci + NBUF - 1, bn)

            for strip in range(CPW // SCH):
                sbase = wid * CPW + strip * SCH
                pltpu.sync_copy(srcs_hbm[g].at[pl.ds(sbase, SCH)], srcv)
                pltpu.sync_copy(dsts_hbm[g].at[pl.ds(sbase, SCH)], dstv)

                for b in range(NBUF - 1):
                    gather(b, b)
                slot(0, 0, first=True)
                for ci in range(1, NBUF):
                    slot(ci, ci % NBUF)

                def body(o, carry):
                    for b in range(NBUF):
                        slot(o * NBUF + b, b)
                    return carry

                lax.fori_loop(1, SCH // NBUF - 1, body, 0)
                for ci in range(SCH - NBUF, SCH):
                    slot(ci, ci % NBUF, fire=ci + NBUF - 1 < SCH)
                scatter_wait(SCH - 1, (SCH - 1) % NBUF)

        fill_zero()
        zero_slice()
        plsc.subcore_barrier()

        for g in range(3):
            edge_phase(g)
            plsc.subcore_barrier()
            pltpu.sync_copy(acc.at[pl.ds(s * RPW, RPW)],
                            p_hbm.at[g, c, pl.ds(s * RPW, RPW)])
            if g < 2:
                fill_zero()
                zero_slice()
            plsc.subcore_barrier()

    return k(h0, h1, h2, srcs[0], srcs[1], srcs[2], dsts[0], dsts[1], dsts[2])


def _epilogue(p, g, x, b):
    def body(p0_ref, p1_ref, x_ref, b_ref, o_ref):
        agg = p0_ref[0, 0] + p1_ref[0, 0] + b_ref[...]
        o_ref[...] = jnp.maximum(agg, 0.0) + x_ref[...]

    return pl.pallas_call(
        body,
        grid=(N // MM_BLK,),
        in_specs=[
            pl.BlockSpec((1, 1, MM_BLK, D), lambda i, g=g: (g, 0, i, 0)),
            pl.BlockSpec((1, 1, MM_BLK, D), lambda i, g=g: (g, 1, i, 0)),
            pl.BlockSpec((MM_BLK, D), lambda i: (i, 0)),
            pl.BlockSpec((1, D), lambda i: (0, 0)),
        ],
        out_specs=pl.BlockSpec((MM_BLK, D), lambda i: (i, 0)),
        out_shape=jax.ShapeDtypeStruct((N, D), jnp.float32),
    )(p, p, x, b)


def _prep_edges(edge_index):
    src = edge_index[0].astype(jnp.int32)
    dst = edge_index[1].astype(jnp.int32)
    pad = jnp.arange(EPAD - E, dtype=jnp.int32)
    pad_src = (pad * 197) % N
    pad_dst = PAD_DST + pad % (ACC_ROWS - N)
    src = jnp.concatenate([src, pad_src])
    dst = jnp.concatenate([dst, pad_dst])
    return src.reshape(NCHUNK, CH), dst.reshape(NCHUNK, CH)


def kernel(h_mf_new, h_bp_new, h_cc_new, mf_edge_index, bp_edge_index,
           cc_edge_index, W_mf, b_mf, W_bp, b_bp, W_cc, b_cc):
    xs = (h_mf_new, h_bp_new, h_cc_new)
    hs = tuple(_matmul(x, w) for x, w in zip(xs, (W_mf, W_bp, W_cc)))

    se, de = zip(*(_prep_edges(e) for e in
                   (mf_edge_index, bp_edge_index, cc_edge_index)))

    p = _sc_scatter(hs[0], hs[1], hs[2], se, de)

    bs = (b_mf, b_bp, b_cc)
    outs = tuple(_epilogue(p, g, xs[g], bs[g].reshape(1, D)) for g in range(3))
    return outs


# speedup vs baseline: 1.0279x; 1.0054x over previous
import functools

import jax
import jax.numpy as jnp
from jax import lax
from jax.experimental import pallas as pl
from jax.experimental.pallas import tpu as pltpu
from jax.experimental.pallas import tpu_sc as plsc

N = 10000
E = 320000
D = 128

NC, NS = 2, 16
NW = NC * NS
CH = 64
CPW = 160
SCH = 40
NBUF = 4
NCHUNK = NW * CPW
NREAL = E // CH
PADC = NCHUNK - NREAL
ACC_ROWS = 10240
PAD_DST = N
RPW = ACC_ROWS // NS

MM_BLK = 1000


def _matmul(x, w):
    def body(x_ref, w_ref, o_ref):
        o_ref[...] = jnp.dot(x_ref[...], w_ref[...],
                             preferred_element_type=jnp.float32)

    return pl.pallas_call(
        body,
        grid=(N // MM_BLK,),
        in_specs=[
            pl.BlockSpec((MM_BLK, D), lambda i: (i, 0)),
            pl.BlockSpec((D, D), lambda i: (0, 0)),
        ],
        out_specs=pl.BlockSpec((MM_BLK, D), lambda i: (i, 0)),
        out_shape=jax.ShapeDtypeStruct((N, D), jnp.float32),
    )(x, w)


def _sc_scatter(h0, h1, h2, srcs, dsts):
    mesh = plsc.VectorSubcoreMesh(core_axis_name="c", subcore_axis_name="s")

    @functools.partial(
        pl.kernel,
        out_type=jax.ShapeDtypeStruct((3, NC, ACC_ROWS, D), jnp.float32),
        mesh=mesh,
        scratch_types=[
            pltpu.VMEM((SCH, CH), jnp.int32),
            pltpu.VMEM((SCH, CH), jnp.int32),
            [pltpu.VMEM((CH, D), jnp.float32) for _ in range(NBUF)],
            pltpu.VMEM_SHARED((ACC_ROWS, D), jnp.float32),
            [pltpu.SemaphoreType.DMA for _ in range(NBUF)],
            [pltpu.SemaphoreType.DMA for _ in range(NBUF)],
        ],
    )
    def k(h0_hbm, h1_hbm, h2_hbm, s0_hbm, s1_hbm, s2_hbm, d0_hbm, d1_hbm,
          d2_hbm, ps_hbm, pd_hbm, p_hbm, srcv, dstv, rows, acc, sems, ssems):
        c = lax.axis_index("c")
        s = lax.axis_index("s")
        wid = c * NS + s
        hs = (h0_hbm, h1_hbm, h2_hbm)
        srcs_hbm = (s0_hbm, s1_hbm, s2_hbm)
        dsts_hbm = (d0_hbm, d1_hbm, d2_hbm)

        def fill_zero():
            def fb(r, carry):
                for j in range(D // 16):
                    rows[0][r, pl.ds(j * 16, 16)] = jnp.zeros((16,), jnp.float32)
                return carry
            lax.fori_loop(0, CH, fb, 0)

        def zero_slice():
            for j in range(RPW // CH):
                pltpu.sync_copy(rows[0], acc.at[pl.ds(s * RPW + j * CH, CH)])

        def edge_phase(g):
            h = hs[g]

            def gather(ci, b):
                pltpu.async_copy(h.at[srcv.at[ci]], rows[b], sems[b])

            def gather_wait(ci, b):
                pltpu.make_async_copy(h.at[srcv.at[ci]], rows[b], sems[b]).wait()

            def scatter(ci, b):
                pltpu.async_copy(rows[b], acc.at[dstv.at[ci]], ssems[b], add=True)

            def scatter_wait(ci, b):
                pltpu.make_async_copy(rows[b], acc.at[dstv.at[ci]], ssems[b]).wait()

            def slot(ci, b, first=False, fire=True):
                gather_wait(ci, b)
                scatter(ci, b)
                bn = (b + NBUF - 1) % NBUF
                if not first:
                    scatter_wait(ci - 1, bn)
                if fire:
                    gather(ci + NBUF - 1, bn)

            for strip in range(CPW // SCH):
                sbase = wid * CPW + strip * SCH

                @pl.when(sbase < NREAL)
                def _():
                    pltpu.sync_copy(srcs_hbm[g].at[pl.ds(sbase, SCH)], srcv)
                    pltpu.sync_copy(dsts_hbm[g].at[pl.ds(sbase, SCH)], dstv)

                @pl.when(sbase >= NREAL)
                def _():
                    pb = sbase - NREAL
                    pltpu.sync_copy(ps_hbm.at[pl.ds(pb, SCH)], srcv)
                    pltpu.sync_copy(pd_hbm.at[pl.ds(pb, SCH)], dstv)

                for b in range(NBUF - 1):
                    gather(b, b)
                slot(0, 0, first=True)
                for ci in range(1, NBUF):
                    slot(ci, ci % NBUF)

                def body(o, carry):
                    for b in range(NBUF):
                        slot(o * NBUF + b, b)
                    return carry

                lax.fori_loop(1, SCH // NBUF - 1, body, 0)
                for ci in range(SCH - NBUF, SCH):
                    slot(ci, ci % NBUF, fire=ci + NBUF - 1 < SCH)
                scatter_wait(SCH - 1, (SCH - 1) % NBUF)

        fill_zero()
        zero_slice()
        plsc.subcore_barrier()

        for g in range(3):
            edge_phase(g)
            plsc.subcore_barrier()
            pltpu.sync_copy(acc.at[pl.ds(s * RPW, RPW)],
                            p_hbm.at[g, c, pl.ds(s * RPW, RPW)])
            if g < 2:
                fill_zero()
                zero_slice()
            plsc.subcore_barrier()

    pad = jnp.arange(PADC * CH, dtype=jnp.int32)
    pad_src = ((pad * 197) % N).reshape(PADC, CH)
    pad_dst = (PAD_DST + pad % (ACC_ROWS - N)).reshape(PADC, CH)

    return k(h0, h1, h2, srcs[0], srcs[1], srcs[2], dsts[0], dsts[1], dsts[2],
             pad_src, pad_dst)


def _epilogue(p, g, x, b):
    def body(p0_ref, p1_ref, x_ref, b_ref, o_ref):
        agg = p0_ref[0, 0] + p1_ref[0, 0] + b_ref[...]
        o_ref[...] = jnp.maximum(agg, 0.0) + x_ref[...]

    return pl.pallas_call(
        body,
        grid=(N // MM_BLK,),
        in_specs=[
            pl.BlockSpec((1, 1, MM_BLK, D), lambda i, g=g: (g, 0, i, 0)),
            pl.BlockSpec((1, 1, MM_BLK, D), lambda i, g=g: (g, 1, i, 0)),
            pl.BlockSpec((MM_BLK, D), lambda i: (i, 0)),
            pl.BlockSpec((1, D), lambda i: (0, 0)),
        ],
        out_specs=pl.BlockSpec((MM_BLK, D), lambda i: (i, 0)),
        out_shape=jax.ShapeDtypeStruct((N, D), jnp.float32),
    )(p, p, x, b)


def _prep_edges(edge_index):
    src = edge_index[0].astype(jnp.int32)
    dst = edge_index[1].astype(jnp.int32)
    return src.reshape(NREAL, CH), dst.reshape(NREAL, CH)


def kernel(h_mf_new, h_bp_new, h_cc_new, mf_edge_index, bp_edge_index,
           cc_edge_index, W_mf, b_mf, W_bp, b_bp, W_cc, b_cc):
    xs = (h_mf_new, h_bp_new, h_cc_new)
    hs = tuple(_matmul(x, w) for x, w in zip(xs, (W_mf, W_bp, W_cc)))

    se, de = zip(*(_prep_edges(e) for e in
                   (mf_edge_index, bp_edge_index, cc_edge_index)))

    p = _sc_scatter(hs[0], hs[1], hs[2], se, de)

    bs = (b_mf, b_bp, b_cc)
    outs = tuple(_epilogue(p, g, xs[g], bs[g].reshape(1, D)) for g in range(3))
    return outs


# speedup vs baseline: 1.1109x; 1.0807x over previous
import functools

import jax
import jax.numpy as jnp
from jax import lax
from jax.experimental import pallas as pl
from jax.experimental.pallas import tpu as pltpu
from jax.experimental.pallas import tpu_sc as plsc

N = 10000
E = 320000
D = 128

NC, NS = 2, 16
NW = NC * NS
CH = 64
CPW = 160
SCH = 40
NBUF = 4
NCHUNK = NW * CPW
NREAL = E // CH
PADC = NCHUNK - NREAL
ACC_ROWS = 10240
PAD_DST = N
RPW = ACC_ROWS // NS

MM_BLK = 1000


def _matmul(x, w):
    def body(x_ref, w_ref, o_ref):
        o_ref[...] = jnp.dot(x_ref[...], w_ref[...],
                             preferred_element_type=jnp.float32)

    return pl.pallas_call(
        body,
        grid=(N // MM_BLK,),
        in_specs=[
            pl.BlockSpec((MM_BLK, D), lambda i: (i, 0)),
            pl.BlockSpec((D, D), lambda i: (0, 0)),
        ],
        out_specs=pl.BlockSpec((MM_BLK, D), lambda i: (i, 0)),
        out_shape=jax.ShapeDtypeStruct((N, D), jnp.float32),
    )(x, w)


def _sc_scatter(h0, h1, h2, es):
    mesh = plsc.VectorSubcoreMesh(core_axis_name="c", subcore_axis_name="s")

    @functools.partial(
        pl.kernel,
        out_type=jax.ShapeDtypeStruct((3, NC, ACC_ROWS, D), jnp.float32),
        mesh=mesh,
        scratch_types=[
            pltpu.VMEM((SCH, CH), jnp.int32),
            pltpu.VMEM((SCH, CH), jnp.int32),
            [pltpu.VMEM((CH, D), jnp.float32) for _ in range(NBUF)],
            pltpu.VMEM_SHARED((ACC_ROWS, D), jnp.float32),
            [pltpu.SemaphoreType.DMA for _ in range(NBUF)],
            [pltpu.SemaphoreType.DMA for _ in range(NBUF)],
        ],
    )
    def k(h0_hbm, h1_hbm, h2_hbm, e0_hbm, e1_hbm, e2_hbm,
          ps_hbm, pd_hbm, p_hbm, srcv, dstv, rows, acc, sems, ssems):
        c = lax.axis_index("c")
        s = lax.axis_index("s")
        wid = c * NS + s
        hs = (h0_hbm, h1_hbm, h2_hbm)
        es_hbm = (e0_hbm, e1_hbm, e2_hbm)

        def fill_zero():
            def fb(r, carry):
                for j in range(D // 16):
                    rows[0][r, pl.ds(j * 16, 16)] = jnp.zeros((16,), jnp.float32)
                return carry
            lax.fori_loop(0, CH, fb, 0)

        def zero_slice():
            for j in range(RPW // CH):
                pltpu.sync_copy(rows[0], acc.at[pl.ds(s * RPW + j * CH, CH)])

        def edge_phase(g):
            h = hs[g]

            def gather(ci, b):
                pltpu.async_copy(h.at[srcv.at[ci]], rows[b], sems[b])

            def gather_wait(ci, b):
                pltpu.make_async_copy(h.at[srcv.at[ci]], rows[b], sems[b]).wait()

            def scatter(ci, b):
                pltpu.async_copy(rows[b], acc.at[dstv.at[ci]], ssems[b], add=True)

            def scatter_wait(ci, b):
                pltpu.make_async_copy(rows[b], acc.at[dstv.at[ci]], ssems[b]).wait()

            def slot(ci, b, first=False, fire=True):
                gather_wait(ci, b)
                scatter(ci, b)
                bn = (b + NBUF - 1) % NBUF
                if not first:
                    scatter_wait(ci - 1, bn)
                if fire:
                    gather(ci + NBUF - 1, bn)

            for strip in range(CPW // SCH):
                sbase = wid * CPW + strip * SCH

                @pl.when(sbase < NREAL)
                def _():
                    pltpu.sync_copy(es_hbm[g].at[0, pl.ds(sbase, SCH)], srcv)
                    pltpu.sync_copy(es_hbm[g].at[1, pl.ds(sbase, SCH)], dstv)

                @pl.when(sbase >= NREAL)
                def _():
                    pb = sbase - NREAL
                    pltpu.sync_copy(ps_hbm.at[pl.ds(pb, SCH)], srcv)
                    pltpu.sync_copy(pd_hbm.at[pl.ds(pb, SCH)], dstv)

                for b in range(NBUF - 1):
                    gather(b, b)
                slot(0, 0, first=True)
                for ci in range(1, NBUF):
                    slot(ci, ci % NBUF)

                def body(o, carry):
                    for b in range(NBUF):
                        slot(o * NBUF + b, b)
                    return carry

                lax.fori_loop(1, SCH // NBUF - 1, body, 0)
                for ci in range(SCH - NBUF, SCH):
                    slot(ci, ci % NBUF, fire=ci + NBUF - 1 < SCH)
                scatter_wait(SCH - 1, (SCH - 1) % NBUF)

        fill_zero()
        zero_slice()
        plsc.subcore_barrier()

        for g in range(3):
            edge_phase(g)
            plsc.subcore_barrier()
            pltpu.sync_copy(acc.at[pl.ds(s * RPW, RPW)],
                            p_hbm.at[g, c, pl.ds(s * RPW, RPW)])
            if g < 2:
                fill_zero()
                zero_slice()
            plsc.subcore_barrier()

    pad = jnp.arange(PADC * CH, dtype=jnp.int32)
    pad_src = ((pad * 197) % N).reshape(PADC, CH)
    pad_dst = (PAD_DST + pad % (ACC_ROWS - N)).reshape(PADC, CH)

    return k(h0, h1, h2, es[0], es[1], es[2], pad_src, pad_dst)


def _epilogue(p, g, x, b):
    def body(p0_ref, p1_ref, x_ref, b_ref, o_ref):
        agg = p0_ref[0, 0] + p1_ref[0, 0] + b_ref[...]
        o_ref[...] = jnp.maximum(agg, 0.0) + x_ref[...]

    return pl.pallas_call(
        body,
        grid=(N // MM_BLK,),
        in_specs=[
            pl.BlockSpec((1, 1, MM_BLK, D), lambda i, g=g: (g, 0, i, 0)),
            pl.BlockSpec((1, 1, MM_BLK, D), lambda i, g=g: (g, 1, i, 0)),
            pl.BlockSpec((MM_BLK, D), lambda i: (i, 0)),
            pl.BlockSpec((1, D), lambda i: (0, 0)),
        ],
        out_specs=pl.BlockSpec((MM_BLK, D), lambda i: (i, 0)),
        out_shape=jax.ShapeDtypeStruct((N, D), jnp.float32),
    )(p, p, x, b)


def _prep_edges(edge_index):
    return edge_index.astype(jnp.int32).reshape(2, NREAL, CH)


def kernel(h_mf_new, h_bp_new, h_cc_new, mf_edge_index, bp_edge_index,
           cc_edge_index, W_mf, b_mf, W_bp, b_bp, W_cc, b_cc):
    xs = (h_mf_new, h_bp_new, h_cc_new)
    hs = tuple(_matmul(x, w) for x, w in zip(xs, (W_mf, W_bp, W_cc)))

    es = tuple(_prep_edges(e) for e in
               (mf_edge_index, bp_edge_index, cc_edge_index))

    p = _sc_scatter(hs[0], hs[1], hs[2], es)

    bs = (b_mf, b_bp, b_cc)
    outs = tuple(_epilogue(p, g, xs[g], bs[g].reshape(1, D)) for g in range(3))
    return outs


# speedup vs baseline: 1.1967x; 1.0772x over previous
import functools

import jax
import jax.numpy as jnp
from jax import lax
from jax.experimental import pallas as pl
from jax.experimental.pallas import tpu as pltpu
from jax.experimental.pallas import tpu_sc as plsc

N = 10000
E = 320000
D = 128

NC, NS = 2, 16
NW = NC * NS
CH = 64
CPW = 160
SCH = 40
NBUF = 4
NCHUNK = NW * CPW
NREAL = E // CH
PADC = NCHUNK - NREAL
ACC_ROWS = 10240
PAD_DST = N
RPW = ACC_ROWS // NS

MM_BLK = 1000


def _matmul(x, w):
    def body(x_ref, w_ref, o_ref):
        o_ref[...] = jnp.dot(x_ref[...], w_ref[...],
                             preferred_element_type=jnp.float32)

    return pl.pallas_call(
        body,
        grid=(N // MM_BLK,),
        in_specs=[
            pl.BlockSpec((MM_BLK, D), lambda i: (i, 0)),
            pl.BlockSpec((D, D), lambda i: (0, 0)),
        ],
        out_specs=pl.BlockSpec((MM_BLK, D), lambda i: (i, 0)),
        out_shape=jax.ShapeDtypeStruct((N, D), jnp.float32),
    )(x, w)


def _sc_scatter(h, e, pad_src, pad_dst):
    mesh = plsc.VectorSubcoreMesh(core_axis_name="c", subcore_axis_name="s")

    @functools.partial(
        pl.kernel,
        out_type=jax.ShapeDtypeStruct((NC, ACC_ROWS, D), jnp.float32),
        mesh=mesh,
        scratch_types=[
            pltpu.VMEM((SCH, CH), jnp.int32),
            pltpu.VMEM((SCH, CH), jnp.int32),
            [pltpu.VMEM((CH, D), jnp.float32) for _ in range(NBUF)],
            pltpu.VMEM_SHARED((ACC_ROWS, D), jnp.float32),
            [pltpu.SemaphoreType.DMA for _ in range(NBUF)],
            [pltpu.SemaphoreType.DMA for _ in range(NBUF)],
        ],
    )
    def k(h_hbm, e_hbm, ps_hbm, pd_hbm, p_hbm,
          srcv, dstv, rows, acc, sems, ssems):
        c = lax.axis_index("c")
        s = lax.axis_index("s")
        wid = c * NS + s

        def fill_zero():
            def fb(r, carry):
                for j in range(D // 16):
                    rows[0][r, pl.ds(j * 16, 16)] = jnp.zeros((16,), jnp.float32)
                return carry
            lax.fori_loop(0, CH, fb, 0)

        def zero_slice():
            for j in range(RPW // CH):
                pltpu.sync_copy(rows[0], acc.at[pl.ds(s * RPW + j * CH, CH)])

        def edge_phase():
            def gather(ci, b):
                pltpu.async_copy(h_hbm.at[srcv.at[ci]], rows[b], sems[b])

            def gather_wait(ci, b):
                pltpu.make_async_copy(h_hbm.at[srcv.at[ci]], rows[b],
                                      sems[b]).wait()

            def scatter(ci, b):
                pltpu.async_copy(rows[b], acc.at[dstv.at[ci]], ssems[b], add=True)

            def scatter_wait(ci, b):
                pltpu.make_async_copy(rows[b], acc.at[dstv.at[ci]], ssems[b]).wait()

            def slot(ci, b, first=False, fire=True):
                gather_wait(ci, b)
                scatter(ci, b)
                bn = (b + NBUF - 1) % NBUF
                if not first:
                    scatter_wait(ci - 1, bn)
                if fire:
                    gather(ci + NBUF - 1, bn)

            for strip in range(CPW // SCH):
                sbase = wid * CPW + strip * SCH

                @pl.when(sbase < NREAL)
                def _():
                    pltpu.sync_copy(e_hbm.at[0, pl.ds(sbase, SCH)], srcv)
                    pltpu.sync_copy(e_hbm.at[1, pl.ds(sbase, SCH)], dstv)

                @pl.when(sbase >= NREAL)
                def _():
                    pb = sbase - NREAL
                    pltpu.sync_copy(ps_hbm.at[pl.ds(pb, SCH)], srcv)
                    pltpu.sync_copy(pd_hbm.at[pl.ds(pb, SCH)], dstv)

                for b in range(NBUF - 1):
                    gather(b, b)
                slot(0, 0, first=True)
                for ci in range(1, NBUF):
                    slot(ci, ci % NBUF)

                def body(o, carry):
                    for b in range(NBUF):
                        slot(o * NBUF + b, b)
                    return carry

                lax.fori_loop(1, SCH // NBUF - 1, body, 0)
                for ci in range(SCH - NBUF, SCH):
                    slot(ci, ci % NBUF, fire=ci + NBUF - 1 < SCH)
                scatter_wait(SCH - 1, (SCH - 1) % NBUF)

        fill_zero()
        zero_slice()
        plsc.subcore_barrier()
        edge_phase()
        plsc.subcore_barrier()
        pltpu.sync_copy(acc.at[pl.ds(s * RPW, RPW)],
                        p_hbm.at[c, pl.ds(s * RPW, RPW)])

    return k(h, e, pad_src, pad_dst)


def _epilogue(p, x, b):
    def body(p0_ref, p1_ref, x_ref, b_ref, o_ref):
        agg = p0_ref[0] + p1_ref[0] + b_ref[...]
        o_ref[...] = jnp.maximum(agg, 0.0) + x_ref[...]

    return pl.pallas_call(
        body,
        grid=(N // MM_BLK,),
        in_specs=[
            pl.BlockSpec((1, MM_BLK, D), lambda i: (0, i, 0)),
            pl.BlockSpec((1, MM_BLK, D), lambda i: (1, i, 0)),
            pl.BlockSpec((MM_BLK, D), lambda i: (i, 0)),
            pl.BlockSpec((1, D), lambda i: (0, 0)),
        ],
        out_specs=pl.BlockSpec((MM_BLK, D), lambda i: (i, 0)),
        out_shape=jax.ShapeDtypeStruct((N, D), jnp.float32),
    )(p, p, x, b)


def _prep_edges(edge_index):
    return edge_index.astype(jnp.int32).reshape(2, NREAL, CH)


def kernel(h_mf_new, h_bp_new, h_cc_new, mf_edge_index, bp_edge_index,
           cc_edge_index, W_mf, b_mf, W_bp, b_bp, W_cc, b_cc):
    xs = (h_mf_new, h_bp_new, h_cc_new)
    hs = tuple(_matmul(x, w) for x, w in zip(xs, (W_mf, W_bp, W_cc)))

    es = tuple(_prep_edges(e) for e in
               (mf_edge_index, bp_edge_index, cc_edge_index))

    pad = jnp.arange(PADC * CH, dtype=jnp.int32)
    pad_src = ((pad * 197) % N).reshape(PADC, CH)
    pad_dst = (PAD_DST + pad % (ACC_ROWS - N)).reshape(PADC, CH)

    ps = tuple(_sc_scatter(hs[g], es[g], pad_src, pad_dst) for g in range(3))

    bs = (b_mf, b_bp, b_cc)
    outs = tuple(_epilogue(ps[g], xs[g], bs[g].reshape(1, D)) for g in range(3))
    return outs


# speedup vs baseline: 1.2319x; 1.0294x over previous
import functools

import jax
import jax.numpy as jnp
from jax import lax
from jax.experimental import pallas as pl
from jax.experimental.pallas import tpu as pltpu
from jax.experimental.pallas import tpu_sc as plsc

N = 10000
E = 320000
D = 128

NC, NS = 2, 16
NW = NC * NS
CH = 64
CPW = 160
SCH = 40
NBUF = 4
NCHUNK = NW * CPW
NREAL = E // CH
PADC = NCHUNK - NREAL
ACC_ROWS = 10240
PAD_DST = N
RPW = ACC_ROWS // NS

MM_BLK = 1000


def _matmul(x, w):
    def body(x_ref, w_ref, o_ref):
        o_ref[...] = jnp.dot(x_ref[...], w_ref[...],
                             preferred_element_type=jnp.float32)

    return pl.pallas_call(
        body,
        grid=(N // MM_BLK,),
        in_specs=[
            pl.BlockSpec((MM_BLK, D), lambda i: (i, 0)),
            pl.BlockSpec((D, D), lambda i: (0, 0)),
        ],
        out_specs=pl.BlockSpec((MM_BLK, D), lambda i: (i, 0)),
        out_shape=jax.ShapeDtypeStruct((N, D), jnp.float32),
    )(x, w)


def _sc_scatter(h, e, pad_src, pad_dst):
    mesh = plsc.VectorSubcoreMesh(core_axis_name="c", subcore_axis_name="s")

    @functools.partial(
        pl.kernel,
        out_type=jax.ShapeDtypeStruct((NC, ACC_ROWS, D), jnp.float32),
        mesh=mesh,
        scratch_types=[
            pltpu.VMEM((SCH, CH), jnp.int32),
            pltpu.VMEM((SCH, CH), jnp.int32),
            [pltpu.VMEM((CH, D), jnp.float32) for _ in range(NBUF)],
            pltpu.VMEM_SHARED((ACC_ROWS, D), jnp.float32),
            [pltpu.SemaphoreType.DMA for _ in range(NBUF)],
            [pltpu.SemaphoreType.DMA for _ in range(NBUF)],
        ],
    )
    def k(h_hbm, e_hbm, ps_hbm, pd_hbm, p_hbm,
          srcv, dstv, rows, acc, sems, ssems):
        c = lax.axis_index("c")
        s = lax.axis_index("s")
        wid = c * NS + s

        ZB = NBUF - 1

        def fill_zero():
            def fb(r, carry):
                for j in range(D // 16):
                    rows[ZB][r, pl.ds(j * 16, 16)] = jnp.zeros((16,),
                                                               jnp.float32)
                return carry
            lax.fori_loop(0, CH, fb, 0)

        def zero_slice():
            for j in range(RPW // CH):
                pltpu.sync_copy(rows[ZB], acc.at[pl.ds(s * RPW + j * CH, CH)])

        def gather(ci, b):
            pltpu.async_copy(h_hbm.at[srcv.at[ci]], rows[b], sems[b])

        def gather_wait(ci, b):
            pltpu.make_async_copy(h_hbm.at[srcv.at[ci]], rows[b],
                                  sems[b]).wait()

        def scatter(ci, b):
            pltpu.async_copy(rows[b], acc.at[dstv.at[ci]], ssems[b], add=True)

        def scatter_wait(ci, b):
            pltpu.make_async_copy(rows[b], acc.at[dstv.at[ci]], ssems[b]).wait()

        def stage(strip):
            sbase = wid * CPW + strip * SCH

            @pl.when(sbase < NREAL)
            def _():
                pltpu.sync_copy(e_hbm.at[0, pl.ds(sbase, SCH)], srcv)
                pltpu.sync_copy(e_hbm.at[1, pl.ds(sbase, SCH)], dstv)

            @pl.when(sbase >= NREAL)
            def _():
                pb = sbase - NREAL
                pltpu.sync_copy(ps_hbm.at[pl.ds(pb, SCH)], srcv)
                pltpu.sync_copy(pd_hbm.at[pl.ds(pb, SCH)], dstv)

        def slot(ci, b, first=False, fire=True):
            gather_wait(ci, b)
            scatter(ci, b)
            bn = (b + NBUF - 1) % NBUF
            if not first:
                scatter_wait(ci - 1, bn)
            if fire:
                gather(ci + NBUF - 1, bn)

        def run_strip(strip, primed=False):
            if not primed:
                stage(strip)
                for b in range(NBUF - 1):
                    gather(b, b)
            slot(0, 0, first=True)
            for ci in range(1, NBUF):
                slot(ci, ci % NBUF)

            def body(o, carry):
                for b in range(NBUF):
                    slot(o * NBUF + b, b)
                return carry

            lax.fori_loop(1, SCH // NBUF - 1, body, 0)
            for ci in range(SCH - NBUF, SCH):
                slot(ci, ci % NBUF, fire=ci + NBUF - 1 < SCH)
            scatter_wait(SCH - 1, (SCH - 1) % NBUF)

        stage(0)
        for b in range(NBUF - 1):
            gather(b, b)
        fill_zero()
        zero_slice()
        plsc.subcore_barrier()
        run_strip(0, primed=True)
        for strip in range(1, CPW // SCH):
            run_strip(strip)
        plsc.subcore_barrier()
        pltpu.sync_copy(acc.at[pl.ds(s * RPW, RPW)],
                        p_hbm.at[c, pl.ds(s * RPW, RPW)])

    return k(h, e, pad_src, pad_dst)


def _epilogue(p, x, b):
    def body(p0_ref, p1_ref, x_ref, b_ref, o_ref):
        agg = p0_ref[0] + p1_ref[0] + b_ref[...]
        o_ref[...] = jnp.maximum(agg, 0.0) + x_ref[...]

    return pl.pallas_call(
        body,
        grid=(N // MM_BLK,),
        in_specs=[
            pl.BlockSpec((1, MM_BLK, D), lambda i: (0, i, 0)),
            pl.BlockSpec((1, MM_BLK, D), lambda i: (1, i, 0)),
            pl.BlockSpec((MM_BLK, D), lambda i: (i, 0)),
            pl.BlockSpec((1, D), lambda i: (0, 0)),
        ],
        out_specs=pl.BlockSpec((MM_BLK, D), lambda i: (i, 0)),
        out_shape=jax.ShapeDtypeStruct((N, D), jnp.float32),
    )(p, p, x, b)


def _prep_edges(edge_index):
    return edge_index.astype(jnp.int32).reshape(2, NREAL, CH)


def kernel(h_mf_new, h_bp_new, h_cc_new, mf_edge_index, bp_edge_index,
           cc_edge_index, W_mf, b_mf, W_bp, b_bp, W_cc, b_cc):
    xs = (h_mf_new, h_bp_new, h_cc_new)
    hs = tuple(_matmul(x, w) for x, w in zip(xs, (W_mf, W_bp, W_cc)))

    es = tuple(_prep_edges(e) for e in
               (mf_edge_index, bp_edge_index, cc_edge_index))

    pad = jnp.arange(PADC * CH, dtype=jnp.int32)
    pad_src = ((pad * 197) % N).reshape(PADC, CH)
    pad_dst = (PAD_DST + pad % (ACC_ROWS - N)).reshape(PADC, CH)

    ps = tuple(_sc_scatter(hs[g], es[g], pad_src, pad_dst) for g in range(3))

    bs = (b_mf, b_bp, b_cc)
    outs = tuple(_epilogue(ps[g], xs[g], bs[g].reshape(1, D)) for g in range(3))
    return outs
